# baseline (speedup 1.0000x reference)
import functools

import jax
import jax.numpy as jnp
from jax import lax
from jax.experimental import pallas as pl
from jax.experimental.pallas import tpu as pltpu
from jax.experimental.pallas import tpu_sc as plsc

B, L, H, V = 1024, 200, 128, 100000
P, T, S, M = 512, 2, 20, 4

NC, NS = 2, 16
NW = NC * NS
NHALF = 2
BH = B // NHALF
BLH = BH * L
ROWS_PER_W = BLH // NW
CPR = 128
NCHUNK = ROWS_PER_W // CPR


@functools.cache
def _sc_gather_call():
    mesh = plsc.VectorSubcoreMesh(core_axis_name="c", subcore_axis_name="s",
                                  num_cores=NC, num_subcores=NS)
    return pl.kernel(
        _sc_gather_body,
        out_type=jax.ShapeDtypeStruct((BLH, H), jnp.float32),
        mesh=mesh,
        scratch_types=[
            pltpu.VMEM((NCHUNK, CPR), jnp.int32),
            pltpu.VMEM((CPR, H), jnp.float32),
            pltpu.VMEM((CPR, H), jnp.float32),
            pltpu.SemaphoreType.DMA,
            pltpu.SemaphoreType.DMA,
        ],
    )


def _sc_gather_body(table_hbm, idx_hbm, out_hbm, idx_v, rows0, rows1, sem0,
                    sem1):
    wid = lax.axis_index("s") * NC + lax.axis_index("c")
    base = wid * ROWS_PER_W
    pltpu.sync_copy(idx_hbm.at[wid], idx_v)
    pltpu.async_copy(table_hbm.at[idx_v.at[0]], rows0, sem0)

    def body(t, _):
        a = 2 * t
        b = a + 1
        pltpu.async_copy(table_hbm.at[idx_v.at[b]], rows1, sem1)
        pltpu.make_async_copy(table_hbm.at[idx_v.at[a]], rows0, sem0).wait()
        pltpu.sync_copy(rows0, out_hbm.at[pl.ds(base + a * CPR, CPR)])

        @pl.when(b + 1 < NCHUNK)
        def _():
            pltpu.async_copy(table_hbm.at[idx_v.at[b + 1]], rows0, sem0)

        pltpu.make_async_copy(table_hbm.at[idx_v.at[b]], rows1, sem1).wait()
        pltpu.sync_copy(rows1, out_hbm.at[pl.ds(base + b * CPR, CPR)])
        return 0

    lax.fori_loop(0, NCHUNK // 2, body, 0)
    if NCHUNK % 2:
        j = NCHUNK - 1
        pltpu.make_async_copy(table_hbm.at[idx_v.at[j]], rows0, sem0).wait()
        pltpu.sync_copy(rows0, out_hbm.at[pl.ds(base + j * CPR, CPR)])


G = 8
STEPS_H = BH // G


def _ln(x, eps, g, b):
    mu = jnp.mean(x, axis=-1, keepdims=True)
    var = jnp.mean((x - mu) ** 2, axis=-1, keepdims=True)
    return (x - mu) * lax.rsqrt(var + eps) * g + b


def _dense_body(g_ref, v_ref, sp_ref, mo_ref, w1_ref, w2_ref, sptab_ref,
                motab_ref, posb_ref, cls_ref, b1_ref,
                g1_ref, be1_ref, g2_ref, be2_ref, out_ref):
    e = g_ref[...].reshape(G * L, H).astype(jnp.bfloat16)
    v2 = v_ref[...].astype(jnp.bfloat16)
    h = (jnp.dot(e, w1_ref[0:H, :].astype(jnp.bfloat16),
                 preferred_element_type=jnp.float32)
         + jnp.dot(v2, w1_ref[H:H + 1, :].astype(jnp.bfloat16),
                   preferred_element_type=jnp.float32)).reshape(G, L, H)
    h = h + b1_ref[...][None]
    h = _ln(h, 1e-5, g1_ref[...][None], be1_ref[...][None])
    h = h * jax.nn.sigmoid(1.702 * h)
    h = jnp.dot(h.reshape(G * L, H).astype(jnp.bfloat16),
                w2_ref[...].astype(jnp.bfloat16),
                preferred_element_type=jnp.float32).reshape(G, L, H)
    body3 = h + posb_ref[...][None]
    out_ref[:, 3:, :] = _ln(body3, 1e-12, g2_ref[...][None], be2_ref[...][None])

    spv = sp_ref[0, 0, :]
    mov = mo_ref[0, 0, :]
    sp_oh = (spv[:, None] == lax.broadcasted_iota(jnp.int32, (1, S), 1)
             ).astype(jnp.float32)
    mo_oh = (mov[:, None] == lax.broadcasted_iota(jnp.int32, (1, M), 1)
             ).astype(jnp.float32)
    sp_rows = jnp.dot(sp_oh, sptab_ref[...], preferred_element_type=jnp.float32)
    mo_rows = jnp.dot(mo_oh, motab_ref[...], preferred_element_type=jnp.float32)
    cls_rows = jnp.broadcast_to(cls_ref[...], (G, H))
    prefix = jnp.concatenate(
        [cls_rows[:, None, :], sp_rows[:, None, :], mo_rows[:, None, :]],
        axis=1)
    out_ref[:, 0:3, :] = _ln(prefix, 1e-12, g2_ref[...][None],
                             be2_ref[...][None])


def _whole(shape):
    n = len(shape)
    return pl.BlockSpec(shape, lambda i: (0,) * n)


def _dense_in_specs():
    return [
        pl.BlockSpec((G, L, H), lambda i: (i, 0, 0)),
        pl.BlockSpec((G * L, 1), lambda i: (i, 0)),
        pl.BlockSpec((1, 1, G), lambda i: (i, 0, 0)),
        pl.BlockSpec((1, 1, G), lambda i: (i, 0, 0)),
        _whole((H + 1, H)),
        _whole((H, H)),
        _whole((S, H)),
        _whole((M, H)),
        _whole((L, H)),
        _whole((1, H)),
        _whole((1, H)),
        _whole((1, H)),
        _whole((1, H)),
        _whole((1, H)),
        _whole((1, H)),
    ]


_dense_call_h0 = pl.pallas_call(
    _dense_body,
    grid=(STEPS_H,),
    in_specs=_dense_in_specs(),
    out_specs=pl.BlockSpec((G, L + 3, H), lambda i: (i, 0, 0)),
    out_shape=jax.ShapeDtypeStruct((B, L + 3, H), jnp.float32),
)

def _dense_body_h1(*refs):
    _dense_body(*refs[:15], refs[16])


_dense_call_h1 = pl.pallas_call(
    _dense_body_h1,
    grid=(STEPS_H,),
    in_specs=_dense_in_specs() + [pl.BlockSpec(memory_space=pl.ANY)],
    out_specs=pl.BlockSpec((G, L + 3, H), lambda i: (i + STEPS_H, 0, 0)),
    out_shape=jax.ShapeDtypeStruct((B, L + 3, H), jnp.float32),
    input_output_aliases={15: 0},
)


def kernel(input_ids, values, species, modality, word_emb, cls_token,
           W1, b1, ln1_g, ln1_b, W2, b2, tte, pos_tab, sp_tab, mo_tab,
           ln2_g, ln2_b):
    sc = _sc_gather_call()
    idx4d = input_ids.reshape(NHALF, NW, NCHUNK, CPR)
    r = lambda a: a.reshape(1, H)
    posb = pos_tab[3:L + 3] + b2[None, :] + tte[0][None, :]
    sptab0 = sp_tab + pos_tab[1][None, :]
    motab0 = mo_tab + pos_tab[2][None, :]
    cls0 = cls_token.reshape(1, H) + pos_tab[0][None, :]
    consts = (W1, W2, sptab0, motab0, posb, cls0, r(b1),
              r(ln1_g), r(ln1_b), r(ln2_g), r(ln2_b))
    vcol = values.reshape(B * L, 1)

    g0 = sc(word_emb, idx4d[0]).reshape(BH, L, H)
    g1 = sc(word_emb, idx4d[1]).reshape(BH, L, H)

    out = _dense_call_h0(
        g0, vcol[:BH * L],
        species[:BH].reshape(STEPS_H, 1, G), modality[:BH].reshape(STEPS_H, 1, G),
        *consts)
    out = _dense_call_h1(
        g1, vcol[BH * L:],
        species[BH:].reshape(STEPS_H, 1, G), modality[BH:].reshape(STEPS_H, 1, G),
        *consts, out)
    return out


# speedup vs baseline: 1.3539x; 1.3539x over previous
import functools

import jax
import jax.numpy as jnp
from jax import lax
from jax.experimental import pallas as pl
from jax.experimental.pallas import tpu as pltpu
from jax.experimental.pallas import tpu_sc as plsc

B, L, H, V = 1024, 200, 128, 100000
P, T, S, M = 512, 2, 20, 4

NC, NS = 2, 16
NW = NC * NS
NHALF = 2
BH = B // NHALF
BLH = BH * L
ROWS_PER_W = BLH // NW
CPR = 128
NCHUNK = ROWS_PER_W // CPR


@functools.cache
def _sc_gather_call():
    mesh = plsc.VectorSubcoreMesh(core_axis_name="c", subcore_axis_name="s",
                                  num_cores=NC, num_subcores=NS)
    return pl.kernel(
        _sc_gather_body,
        out_type=jax.ShapeDtypeStruct((BLH, H), jnp.float32),
        mesh=mesh,
        scratch_types=[
            pltpu.VMEM((NCHUNK, CPR), jnp.int32),
            pltpu.VMEM((CPR, H), jnp.float32),
            pltpu.VMEM((CPR, H), jnp.float32),
            pltpu.SemaphoreType.DMA,
            pltpu.SemaphoreType.DMA,
        ],
    )


def _sc_gather_body(table_hbm, idx_hbm, out_hbm, idx_v, rows0, rows1, sem0,
                    sem1):
    wid = lax.axis_index("s") * NC + lax.axis_index("c")
    base = wid * ROWS_PER_W
    pltpu.sync_copy(idx_hbm.at[wid], idx_v)
    pltpu.async_copy(table_hbm.at[idx_v.at[0]], rows0, sem0)

    def body(t, _):
        a = 2 * t
        b = a + 1
        pltpu.async_copy(table_hbm.at[idx_v.at[b]], rows1, sem1)
        pltpu.make_async_copy(table_hbm.at[idx_v.at[a]], rows0, sem0).wait()
        pltpu.sync_copy(rows0, out_hbm.at[pl.ds(base + a * CPR, CPR)])

        @pl.when(b + 1 < NCHUNK)
        def _():
            pltpu.async_copy(table_hbm.at[idx_v.at[b + 1]], rows0, sem0)

        pltpu.make_async_copy(table_hbm.at[idx_v.at[b]], rows1, sem1).wait()
        pltpu.sync_copy(rows1, out_hbm.at[pl.ds(base + b * CPR, CPR)])
        return 0

    lax.fori_loop(0, NCHUNK // 2, body, 0)
    if NCHUNK % 2:
        j = NCHUNK - 1
        pltpu.make_async_copy(table_hbm.at[idx_v.at[j]], rows0, sem0).wait()
        pltpu.sync_copy(rows0, out_hbm.at[pl.ds(base + j * CPR, CPR)])


G = 8
STEPS_H = BH // G


def _ln(x, eps, g, b):
    mu = jnp.mean(x, axis=-1, keepdims=True)
    var = jnp.mean((x - mu) ** 2, axis=-1, keepdims=True)
    return (x - mu) * lax.rsqrt(var + eps) * g + b


def _dense_body(g_ref, v_ref, sp_ref, mo_ref, w1_ref, w2_ref, sptab_ref,
                motab_ref, posb_ref, cls_ref, b1_ref,
                g1_ref, be1_ref, g2_ref, be2_ref, out_ref):
    e = g_ref[...].reshape(G * L, H).astype(jnp.bfloat16)
    h = jnp.dot(e, w1_ref[0:H, :].astype(jnp.bfloat16),
                preferred_element_type=jnp.float32).reshape(G, L, H)
    h = (h + v_ref[...][..., None] * w1_ref[H:H + 1, :][None]
         + b1_ref[...][None])
    h = _ln(h, 1e-5, g1_ref[...][None], be1_ref[...][None])
    h = h * jax.nn.sigmoid(1.702 * h)
    h = jnp.dot(h.reshape(G * L, H).astype(jnp.bfloat16),
                w2_ref[...].astype(jnp.bfloat16),
                preferred_element_type=jnp.float32).reshape(G, L, H)
    body3 = h + posb_ref[...][None]
    out_ref[:, 3:, :] = _ln(body3, 1e-12, g2_ref[...][None], be2_ref[...][None])

    spv = sp_ref[0, 0, :]
    mov = mo_ref[0, 0, :]
    sp_oh = (spv[:, None] == lax.broadcasted_iota(jnp.int32, (1, S), 1)
             ).astype(jnp.float32)
    mo_oh = (mov[:, None] == lax.broadcasted_iota(jnp.int32, (1, M), 1)
             ).astype(jnp.float32)
    sp_rows = jnp.dot(sp_oh, sptab_ref[...], preferred_element_type=jnp.float32)
    mo_rows = jnp.dot(mo_oh, motab_ref[...], preferred_element_type=jnp.float32)
    cls_rows = jnp.broadcast_to(cls_ref[...], (G, H))
    prefix = jnp.concatenate(
        [cls_rows[:, None, :], sp_rows[:, None, :], mo_rows[:, None, :]],
        axis=1)
    out_ref[:, 0:3, :] = _ln(prefix, 1e-12, g2_ref[...][None],
                             be2_ref[...][None])


def _whole(shape):
    n = len(shape)
    return pl.BlockSpec(shape, lambda i: (0,) * n)


def _dense_in_specs():
    return [
        pl.BlockSpec((G, L, H), lambda i: (i, 0, 0)),
        pl.BlockSpec((G, L), lambda i: (i, 0)),
        pl.BlockSpec((1, 1, G), lambda i: (i, 0, 0)),
        pl.BlockSpec((1, 1, G), lambda i: (i, 0, 0)),
        _whole((H + 1, H)),
        _whole((H, H)),
        _whole((S, H)),
        _whole((M, H)),
        _whole((L, H)),
        _whole((1, H)),
        _whole((1, H)),
        _whole((1, H)),
        _whole((1, H)),
        _whole((1, H)),
        _whole((1, H)),
    ]


_dense_call_h0 = pl.pallas_call(
    _dense_body,
    grid=(STEPS_H,),
    in_specs=_dense_in_specs(),
    out_specs=pl.BlockSpec((G, L + 3, H), lambda i: (i, 0, 0)),
    out_shape=jax.ShapeDtypeStruct((B, L + 3, H), jnp.float32),
)

def _dense_body_h1(*refs):
    _dense_body(*refs[:15], refs[16])


_dense_call_h1 = pl.pallas_call(
    _dense_body_h1,
    grid=(STEPS_H,),
    in_specs=_dense_in_specs() + [pl.BlockSpec(memory_space=pl.ANY)],
    out_specs=pl.BlockSpec((G, L + 3, H), lambda i: (i + STEPS_H, 0, 0)),
    out_shape=jax.ShapeDtypeStruct((B, L + 3, H), jnp.float32),
    input_output_aliases={15: 0},
)


def kernel(input_ids, values, species, modality, word_emb, cls_token,
           W1, b1, ln1_g, ln1_b, W2, b2, tte, pos_tab, sp_tab, mo_tab,
           ln2_g, ln2_b):
    sc = _sc_gather_call()
    idx4d = input_ids.reshape(NHALF, NW, NCHUNK, CPR)
    r = lambda a: a.reshape(1, H)
    posb = pos_tab[3:L + 3] + b2[None, :] + tte[0][None, :]
    sptab0 = sp_tab + pos_tab[1][None, :]
    motab0 = mo_tab + pos_tab[2][None, :]
    cls0 = cls_token.reshape(1, H) + pos_tab[0][None, :]
    consts = (W1, W2, sptab0, motab0, posb, cls0, r(b1),
              r(ln1_g), r(ln1_b), r(ln2_g), r(ln2_b))
    g0 = sc(word_emb, idx4d[0]).reshape(BH, L, H)
    g1 = sc(word_emb, idx4d[1]).reshape(BH, L, H)

    out = _dense_call_h0(
        g0, values[:BH],
        species[:BH].reshape(STEPS_H, 1, G), modality[:BH].reshape(STEPS_H, 1, G),
        *consts)
    out = _dense_call_h1(
        g1, values[BH:],
        species[BH:].reshape(STEPS_H, 1, G), modality[BH:].reshape(STEPS_H, 1, G),
        *consts, out)
    return out


# speedup vs baseline: 1.3901x; 1.0267x over previous
import functools

import jax
import jax.numpy as jnp
from jax import lax
from jax.experimental import pallas as pl
from jax.experimental.pallas import tpu as pltpu
from jax.experimental.pallas import tpu_sc as plsc

B, L, H, V = 1024, 200, 128, 100000
P, T, S, M = 512, 2, 20, 4

NC, NS = 2, 16
NW = NC * NS
NHALF = 2
BH = B // NHALF
BLH = BH * L
ROWS_PER_W = BLH // NW
CPR = 128
NCHUNK = ROWS_PER_W // CPR


@functools.cache
def _sc_gather_call():
    mesh = plsc.VectorSubcoreMesh(core_axis_name="c", subcore_axis_name="s",
                                  num_cores=NC, num_subcores=NS)
    return pl.kernel(
        _sc_gather_body,
        out_type=jax.ShapeDtypeStruct((BLH, H), jnp.float32),
        mesh=mesh,
        scratch_types=[
            pltpu.VMEM((NCHUNK, CPR), jnp.int32),
            pltpu.VMEM((CPR, H), jnp.float32),
            pltpu.VMEM((CPR, H), jnp.float32),
            pltpu.SemaphoreType.DMA,
            pltpu.SemaphoreType.DMA,
        ],
    )


def _sc_gather_body(table_hbm, idx_hbm, out_hbm, idx_v, rows0, rows1, sem0,
                    sem1):
    wid = lax.axis_index("s") * NC + lax.axis_index("c")
    base = wid * ROWS_PER_W
    pltpu.sync_copy(idx_hbm.at[wid], idx_v)
    pltpu.async_copy(table_hbm.at[idx_v.at[0]], rows0, sem0)

    def body(t, _):
        a = 2 * t
        b = a + 1
        pltpu.async_copy(table_hbm.at[idx_v.at[b]], rows1, sem1)
        pltpu.make_async_copy(table_hbm.at[idx_v.at[a]], rows0, sem0).wait()
        pltpu.sync_copy(rows0, out_hbm.at[pl.ds(base + a * CPR, CPR)])

        @pl.when(b + 1 < NCHUNK)
        def _():
            pltpu.async_copy(table_hbm.at[idx_v.at[b + 1]], rows0, sem0)

        pltpu.make_async_copy(table_hbm.at[idx_v.at[b]], rows1, sem1).wait()
        pltpu.sync_copy(rows1, out_hbm.at[pl.ds(base + b * CPR, CPR)])
        return 0

    lax.fori_loop(0, NCHUNK // 2, body, 0)
    if NCHUNK % 2:
        j = NCHUNK - 1
        pltpu.make_async_copy(table_hbm.at[idx_v.at[j]], rows0, sem0).wait()
        pltpu.sync_copy(rows0, out_hbm.at[pl.ds(base + j * CPR, CPR)])


G = 8
STEPS_H = BH // G


def _ln_mxu(x2d, eps, g, b):
    jones = jnp.full((H, H), 1.0 / H, dtype=jnp.bfloat16)
    mu = jnp.dot(x2d.astype(jnp.bfloat16), jones,
                 preferred_element_type=jnp.float32)
    d = x2d - mu
    db = d.astype(jnp.bfloat16)
    var = jnp.dot(db * db, jones, preferred_element_type=jnp.float32)
    return d * lax.rsqrt(var + eps) * g + b


def _dense_body(g_ref, v_ref, sp_ref, mo_ref, w1_ref, w2_ref, sptab_ref,
                motab_ref, posb_ref, cls_ref, b1_ref,
                g1_ref, be1_ref, g2_ref, be2_ref, out_ref):
    e = g_ref[...].reshape(G * L, H).astype(jnp.bfloat16)
    h = jnp.dot(e, w1_ref[0:H, :].astype(jnp.bfloat16),
                preferred_element_type=jnp.float32).reshape(G, L, H)
    h = (h + v_ref[...][..., None] * w1_ref[H:H + 1, :][None]
         + b1_ref[...][None])
    h = _ln_mxu(h.reshape(G * L, H), 1e-5, g1_ref[...], be1_ref[...])
    h = h.astype(jnp.bfloat16)
    h = h * jax.nn.sigmoid(jnp.bfloat16(1.702) * h)
    h = jnp.dot(h, w2_ref[...].astype(jnp.bfloat16),
                preferred_element_type=jnp.float32).reshape(G, L, H)
    body3 = h + posb_ref[...][None]
    out_ref[:, 3:, :] = _ln_mxu(body3.reshape(G * L, H), 1e-12, g2_ref[...],
                                be2_ref[...]).reshape(G, L, H)

    spv = sp_ref[0, 0, :]
    mov = mo_ref[0, 0, :]
    sp_oh = (spv[:, None] == lax.broadcasted_iota(jnp.int32, (1, S), 1)
             ).astype(jnp.float32)
    mo_oh = (mov[:, None] == lax.broadcasted_iota(jnp.int32, (1, M), 1)
             ).astype(jnp.float32)
    sp_rows = jnp.dot(sp_oh, sptab_ref[...], preferred_element_type=jnp.float32)
    mo_rows = jnp.dot(mo_oh, motab_ref[...], preferred_element_type=jnp.float32)
    cls_rows = jnp.broadcast_to(cls_ref[...], (G, H))
    prefix = jnp.concatenate(
        [cls_rows[:, None, :], sp_rows[:, None, :], mo_rows[:, None, :]],
        axis=1)
    mu = jnp.mean(prefix, axis=-1, keepdims=True)
    var = jnp.mean((prefix - mu) ** 2, axis=-1, keepdims=True)
    out_ref[:, 0:3, :] = ((prefix - mu) * lax.rsqrt(var + 1e-12)
                          * g2_ref[...][None] + be2_ref[...][None])


def _whole(shape):
    n = len(shape)
    return pl.BlockSpec(shape, lambda i: (0,) * n)


def _dense_in_specs():
    return [
        pl.BlockSpec((G, L, H), lambda i: (i, 0, 0)),
        pl.BlockSpec((G, L), lambda i: (i, 0)),
        pl.BlockSpec((1, 1, G), lambda i: (i, 0, 0)),
        pl.BlockSpec((1, 1, G), lambda i: (i, 0, 0)),
        _whole((H + 1, H)),
        _whole((H, H)),
        _whole((S, H)),
        _whole((M, H)),
        _whole((L, H)),
        _whole((1, H)),
        _whole((1, H)),
        _whole((1, H)),
        _whole((1, H)),
        _whole((1, H)),
        _whole((1, H)),
    ]


_dense_call_h0 = pl.pallas_call(
    _dense_body,
    grid=(STEPS_H,),
    in_specs=_dense_in_specs(),
    out_specs=pl.BlockSpec((G, L + 3, H), lambda i: (i, 0, 0)),
    out_shape=jax.ShapeDtypeStruct((B, L + 3, H), jnp.float32),
)

def _dense_body_h1(*refs):
    _dense_body(*refs[:15], refs[16])


_dense_call_h1 = pl.pallas_call(
    _dense_body_h1,
    grid=(STEPS_H,),
    in_specs=_dense_in_specs() + [pl.BlockSpec(memory_space=pl.ANY)],
    out_specs=pl.BlockSpec((G, L + 3, H), lambda i: (i + STEPS_H, 0, 0)),
    out_shape=jax.ShapeDtypeStruct((B, L + 3, H), jnp.float32),
    input_output_aliases={15: 0},
)


def kernel(input_ids, values, species, modality, word_emb, cls_token,
           W1, b1, ln1_g, ln1_b, W2, b2, tte, pos_tab, sp_tab, mo_tab,
           ln2_g, ln2_b):
    sc = _sc_gather_call()
    idx4d = input_ids.reshape(NHALF, NW, NCHUNK, CPR)
    r = lambda a: a.reshape(1, H)
    posb = pos_tab[3:L + 3] + b2[None, :] + tte[0][None, :]
    sptab0 = sp_tab + pos_tab[1][None, :]
    motab0 = mo_tab + pos_tab[2][None, :]
    cls0 = cls_token.reshape(1, H) + pos_tab[0][None, :]
    consts = (W1, W2, sptab0, motab0, posb, cls0, r(b1),
              r(ln1_g), r(ln1_b), r(ln2_g), r(ln2_b))
    g0 = sc(word_emb, idx4d[0]).reshape(BH, L, H)
    g1 = sc(word_emb, idx4d[1]).reshape(BH, L, H)

    out = _dense_call_h0(
        g0, values[:BH],
        species[:BH].reshape(STEPS_H, 1, G), modality[:BH].reshape(STEPS_H, 1, G),
        *consts)
    out = _dense_call_h1(
        g1, values[BH:],
        species[BH:].reshape(STEPS_H, 1, G), modality[BH:].reshape(STEPS_H, 1, G),
        *consts, out)
    return out


# speedup vs baseline: 1.6115x; 1.1593x over previous
import functools

import jax
import jax.numpy as jnp
from jax import lax
from jax.experimental import pallas as pl
from jax.experimental.pallas import tpu as pltpu
from jax.experimental.pallas import tpu_sc as plsc

B, L, H, V = 1024, 200, 128, 100000
P, T, S, M = 512, 2, 20, 4

NC, NS = 2, 16
NW = NC * NS
NHALF = 2
BH = B // NHALF
BLH = BH * L
ROWS_PER_W = BLH // NW
CPR = 128
NCHUNK = ROWS_PER_W // CPR


@functools.cache
def _sc_gather_call():
    mesh = plsc.VectorSubcoreMesh(core_axis_name="c", subcore_axis_name="s",
                                  num_cores=NC, num_subcores=NS)
    return pl.kernel(
        _sc_gather_body,
        out_type=jax.ShapeDtypeStruct((BLH, H), jnp.float32),
        mesh=mesh,
        scratch_types=[
            pltpu.VMEM((NCHUNK, CPR), jnp.int32),
            pltpu.VMEM((CPR, H), jnp.float32),
            pltpu.VMEM((CPR, H), jnp.float32),
            pltpu.SemaphoreType.DMA,
            pltpu.SemaphoreType.DMA,
        ],
    )


def _sc_gather_body(table_hbm, idx_hbm, out_hbm, idx_v, rows0, rows1, sem0,
                    sem1):
    wid = lax.axis_index("s") * NC + lax.axis_index("c")
    base = wid * ROWS_PER_W
    pltpu.sync_copy(idx_hbm.at[wid], idx_v)
    pltpu.async_copy(table_hbm.at[idx_v.at[0]], rows0, sem0)

    def body(t, _):
        a = 2 * t
        b = a + 1
        pltpu.async_copy(table_hbm.at[idx_v.at[b]], rows1, sem1)
        pltpu.make_async_copy(table_hbm.at[idx_v.at[a]], rows0, sem0).wait()
        pltpu.sync_copy(rows0, out_hbm.at[pl.ds(base + a * CPR, CPR)])

        @pl.when(b + 1 < NCHUNK)
        def _():
            pltpu.async_copy(table_hbm.at[idx_v.at[b + 1]], rows0, sem0)

        pltpu.make_async_copy(table_hbm.at[idx_v.at[b]], rows1, sem1).wait()
        pltpu.sync_copy(rows1, out_hbm.at[pl.ds(base + b * CPR, CPR)])
        return 0

    lax.fori_loop(0, NCHUNK // 2, body, 0)
    if NCHUNK % 2:
        j = NCHUNK - 1
        pltpu.make_async_copy(table_hbm.at[idx_v.at[j]], rows0, sem0).wait()
        pltpu.sync_copy(rows0, out_hbm.at[pl.ds(base + j * CPR, CPR)])


G = 16
STEPS_H = BH // G


def _ln_mxu(x2d, eps, g, b):
    jones = jnp.full((H, H), 1.0 / H, dtype=jnp.bfloat16)
    mu = jnp.dot(x2d.astype(jnp.bfloat16), jones,
                 preferred_element_type=jnp.float32)
    d = x2d - mu
    db = d.astype(jnp.bfloat16)
    var = jnp.dot(db * db, jones, preferred_element_type=jnp.float32)
    return d * lax.rsqrt(var + eps) * g + b


def _dense_body(g_ref, v_ref, sp_ref, mo_ref, w1_ref, w2_ref, sptab_ref,
                motab_ref, posb_ref, cls_ref, b1_ref,
                g1_ref, be1_ref, g2_ref, be2_ref, out_ref):
    e = g_ref[...].reshape(G * L, H).astype(jnp.bfloat16)
    h = jnp.dot(e, w1_ref[0:H, :].astype(jnp.bfloat16),
                preferred_element_type=jnp.float32).reshape(G, L, H)
    h = (h + v_ref[...][..., None] * w1_ref[H:H + 1, :][None]
         + b1_ref[...][None])
    h = _ln_mxu(h.reshape(G * L, H), 1e-5, g1_ref[...], be1_ref[...])
    h = h.astype(jnp.bfloat16)
    h = h * jax.nn.sigmoid(jnp.bfloat16(1.702) * h)
    h = jnp.dot(h, w2_ref[...].astype(jnp.bfloat16),
                preferred_element_type=jnp.float32).reshape(G, L, H)
    body3 = h + posb_ref[...][None]
    out_ref[:, 3:, :] = _ln_mxu(body3.reshape(G * L, H), 1e-12, g2_ref[...],
                                be2_ref[...]).reshape(G, L, H)

    spv = sp_ref[0, 0, :]
    mov = mo_ref[0, 0, :]
    sp_oh = (spv[:, None] == lax.broadcasted_iota(jnp.int32, (1, S), 1)
             ).astype(jnp.float32)
    mo_oh = (mov[:, None] == lax.broadcasted_iota(jnp.int32, (1, M), 1)
             ).astype(jnp.float32)
    sp_rows = jnp.dot(sp_oh, sptab_ref[...], preferred_element_type=jnp.float32)
    mo_rows = jnp.dot(mo_oh, motab_ref[...], preferred_element_type=jnp.float32)
    cls_rows = jnp.broadcast_to(cls_ref[...], (G, H))
    prefix = jnp.concatenate(
        [cls_rows[:, None, :], sp_rows[:, None, :], mo_rows[:, None, :]],
        axis=1)
    mu = jnp.mean(prefix, axis=-1, keepdims=True)
    var = jnp.mean((prefix - mu) ** 2, axis=-1, keepdims=True)
    out_ref[:, 0:3, :] = ((prefix - mu) * lax.rsqrt(var + 1e-12)
                          * g2_ref[...][None] + be2_ref[...][None])


def _whole(shape):
    n = len(shape)
    return pl.BlockSpec(shape, lambda i: (0,) * n)


def _dense_in_specs():
    return [
        pl.BlockSpec((G, L, H), lambda i: (i, 0, 0)),
        pl.BlockSpec((G, L), lambda i: (i, 0)),
        pl.BlockSpec((1, 1, G), lambda i: (i, 0, 0)),
        pl.BlockSpec((1, 1, G), lambda i: (i, 0, 0)),
        _whole((H + 1, H)),
        _whole((H, H)),
        _whole((S, H)),
        _whole((M, H)),
        _whole((L, H)),
        _whole((1, H)),
        _whole((1, H)),
        _whole((1, H)),
        _whole((1, H)),
        _whole((1, H)),
        _whole((1, H)),
    ]


_dense_call_h0 = pl.pallas_call(
    _dense_body,
    grid=(STEPS_H,),
    in_specs=_dense_in_specs(),
    out_specs=pl.BlockSpec((G, L + 3, H), lambda i: (i, 0, 0)),
    out_shape=jax.ShapeDtypeStruct((B, L + 3, H), jnp.float32),
)

def _dense_body_h1(*refs):
    _dense_body(*refs[:15], refs[16])


_dense_call_h1 = pl.pallas_call(
    _dense_body_h1,
    grid=(STEPS_H,),
    in_specs=_dense_in_specs() + [pl.BlockSpec(memory_space=pl.ANY)],
    out_specs=pl.BlockSpec((G, L + 3, H), lambda i: (i + STEPS_H, 0, 0)),
    out_shape=jax.ShapeDtypeStruct((B, L + 3, H), jnp.float32),
    input_output_aliases={15: 0},
)


def kernel(input_ids, values, species, modality, word_emb, cls_token,
           W1, b1, ln1_g, ln1_b, W2, b2, tte, pos_tab, sp_tab, mo_tab,
           ln2_g, ln2_b):
    sc = _sc_gather_call()
    idx4d = input_ids.reshape(NHALF, NW, NCHUNK, CPR)
    r = lambda a: a.reshape(1, H)
    posb = pos_tab[3:L + 3] + b2[None, :] + tte[0][None, :]
    sptab0 = sp_tab + pos_tab[1][None, :]
    motab0 = mo_tab + pos_tab[2][None, :]
    cls0 = cls_token.reshape(1, H) + pos_tab[0][None, :]
    consts = (W1, W2, sptab0, motab0, posb, cls0, r(b1),
              r(ln1_g), r(ln1_b), r(ln2_g), r(ln2_b))
    g0 = sc(word_emb, idx4d[0]).reshape(BH, L, H)
    g1 = sc(word_emb, idx4d[1]).reshape(BH, L, H)

    out = _dense_call_h0(
        g0, values[:BH],
        species[:BH].reshape(STEPS_H, 1, G), modality[:BH].reshape(STEPS_H, 1, G),
        *consts)
    out = _dense_call_h1(
        g1, values[BH:],
        species[BH:].reshape(STEPS_H, 1, G), modality[BH:].reshape(STEPS_H, 1, G),
        *consts, out)
    return out


# speedup vs baseline: 1.7004x; 1.0552x over previous
import functools

import jax
import jax.numpy as jnp
from jax import lax
from jax.experimental import pallas as pl
from jax.experimental.pallas import tpu as pltpu
from jax.experimental.pallas import tpu_sc as plsc

B, L, H, V = 1024, 200, 128, 100000
P, T, S, M = 512, 2, 20, 4

NC, NS = 2, 16
NW = NC * NS
NHALF = 2
BH = B // NHALF
BLH = BH * L
ROWS_PER_W = BLH // NW
CPR = 128
NCHUNK = ROWS_PER_W // CPR


@functools.cache
def _sc_gather_call():
    mesh = plsc.VectorSubcoreMesh(core_axis_name="c", subcore_axis_name="s",
                                  num_cores=NC, num_subcores=NS)
    return pl.kernel(
        _sc_gather_body,
        out_type=jax.ShapeDtypeStruct((BLH, H), jnp.float32),
        mesh=mesh,
        scratch_types=[
            pltpu.VMEM((NCHUNK, CPR), jnp.int32),
            pltpu.VMEM((CPR, H), jnp.float32),
            pltpu.VMEM((CPR, H), jnp.float32),
            pltpu.SemaphoreType.DMA,
            pltpu.SemaphoreType.DMA,
        ],
    )


def _sc_gather_body(table_hbm, idx_hbm, out_hbm, idx_v, rows0, rows1, sem0,
                    sem1):
    wid = lax.axis_index("s") * NC + lax.axis_index("c")
    base = wid * ROWS_PER_W
    pltpu.sync_copy(idx_hbm.at[wid], idx_v)
    pltpu.async_copy(table_hbm.at[idx_v.at[0]], rows0, sem0)

    def body(t, _):
        a = 2 * t
        b = a + 1
        pltpu.async_copy(table_hbm.at[idx_v.at[b]], rows1, sem1)
        pltpu.make_async_copy(table_hbm.at[idx_v.at[a]], rows0, sem0).wait()
        pltpu.sync_copy(rows0, out_hbm.at[pl.ds(base + a * CPR, CPR)])

        @pl.when(b + 1 < NCHUNK)
        def _():
            pltpu.async_copy(table_hbm.at[idx_v.at[b + 1]], rows0, sem0)

        pltpu.make_async_copy(table_hbm.at[idx_v.at[b]], rows1, sem1).wait()
        pltpu.sync_copy(rows1, out_hbm.at[pl.ds(base + b * CPR, CPR)])
        return 0

    lax.fori_loop(0, NCHUNK // 2, body, 0)
    if NCHUNK % 2:
        j = NCHUNK - 1
        pltpu.make_async_copy(table_hbm.at[idx_v.at[j]], rows0, sem0).wait()
        pltpu.sync_copy(rows0, out_hbm.at[pl.ds(base + j * CPR, CPR)])


G = 32
STEPS_H = BH // G


def _ln_mxu(x2d, eps, g, b):
    jones = jnp.full((H, H), 1.0 / H, dtype=jnp.bfloat16)
    mu = jnp.dot(x2d.astype(jnp.bfloat16), jones,
                 preferred_element_type=jnp.float32)
    d = x2d - mu
    db = d.astype(jnp.bfloat16)
    var = jnp.dot(db * db, jones, preferred_element_type=jnp.float32)
    return d * lax.rsqrt(var + eps) * g + b


def _dense_body(g_ref, v_ref, sp_ref, mo_ref, w1_ref, w2_ref, sptab_ref,
                motab_ref, posb_ref, cls_ref, b1_ref,
                g1_ref, be1_ref, g2_ref, be2_ref, out_ref):
    e = g_ref[...].reshape(G * L, H).astype(jnp.bfloat16)
    h = jnp.dot(e, w1_ref[0:H, :].astype(jnp.bfloat16),
                preferred_element_type=jnp.float32).reshape(G, L, H)
    h = (h + v_ref[...][..., None] * w1_ref[H:H + 1, :][None]
         + b1_ref[...][None])
    h = _ln_mxu(h.reshape(G * L, H), 1e-5, g1_ref[...], be1_ref[...])
    h = h.astype(jnp.bfloat16)
    h = h * jax.nn.sigmoid(jnp.bfloat16(1.702) * h)
    h = jnp.dot(h, w2_ref[...].astype(jnp.bfloat16),
                preferred_element_type=jnp.float32).reshape(G, L, H)
    body3 = h + posb_ref[...][None]
    out_ref[:, 3:, :] = _ln_mxu(body3.reshape(G * L, H), 1e-12, g2_ref[...],
                                be2_ref[...]).reshape(G, L, H)

    spv = sp_ref[0, 0, :]
    mov = mo_ref[0, 0, :]
    sp_oh = (spv[:, None] == lax.broadcasted_iota(jnp.int32, (1, S), 1)
             ).astype(jnp.float32)
    mo_oh = (mov[:, None] == lax.broadcasted_iota(jnp.int32, (1, M), 1)
             ).astype(jnp.float32)
    sp_rows = jnp.dot(sp_oh, sptab_ref[...], preferred_element_type=jnp.float32)
    mo_rows = jnp.dot(mo_oh, motab_ref[...], preferred_element_type=jnp.float32)
    cls_rows = jnp.broadcast_to(cls_ref[...], (G, H))
    prefix = jnp.concatenate(
        [cls_rows[:, None, :], sp_rows[:, None, :], mo_rows[:, None, :]],
        axis=1)
    mu = jnp.mean(prefix, axis=-1, keepdims=True)
    var = jnp.mean((prefix - mu) ** 2, axis=-1, keepdims=True)
    out_ref[:, 0:3, :] = ((prefix - mu) * lax.rsqrt(var + 1e-12)
                          * g2_ref[...][None] + be2_ref[...][None])


def _whole(shape):
    n = len(shape)
    return pl.BlockSpec(shape, lambda i: (0,) * n)


def _dense_in_specs():
    return [
        pl.BlockSpec((G, L, H), lambda i: (i, 0, 0)),
        pl.BlockSpec((G, L), lambda i: (i, 0)),
        pl.BlockSpec((1, 1, G), lambda i: (i, 0, 0)),
        pl.BlockSpec((1, 1, G), lambda i: (i, 0, 0)),
        _whole((H + 1, H)),
        _whole((H, H)),
        _whole((S, H)),
        _whole((M, H)),
        _whole((L, H)),
        _whole((1, H)),
        _whole((1, H)),
        _whole((1, H)),
        _whole((1, H)),
        _whole((1, H)),
        _whole((1, H)),
    ]


_dense_call_h0 = pl.pallas_call(
    _dense_body,
    grid=(STEPS_H,),
    in_specs=_dense_in_specs(),
    out_specs=pl.BlockSpec((G, L + 3, H), lambda i: (i, 0, 0)),
    out_shape=jax.ShapeDtypeStruct((B, L + 3, H), jnp.float32),
)

def _dense_body_h1(*refs):
    _dense_body(*refs[:15], refs[16])


_dense_call_h1 = pl.pallas_call(
    _dense_body_h1,
    grid=(STEPS_H,),
    in_specs=_dense_in_specs() + [pl.BlockSpec(memory_space=pl.ANY)],
    out_specs=pl.BlockSpec((G, L + 3, H), lambda i: (i + STEPS_H, 0, 0)),
    out_shape=jax.ShapeDtypeStruct((B, L + 3, H), jnp.float32),
    input_output_aliases={15: 0},
)


def kernel(input_ids, values, species, modality, word_emb, cls_token,
           W1, b1, ln1_g, ln1_b, W2, b2, tte, pos_tab, sp_tab, mo_tab,
           ln2_g, ln2_b):
    sc = _sc_gather_call()
    idx4d = input_ids.reshape(NHALF, NW, NCHUNK, CPR)
    r = lambda a: a.reshape(1, H)
    posb = pos_tab[3:L + 3] + b2[None, :] + tte[0][None, :]
    sptab0 = sp_tab + pos_tab[1][None, :]
    motab0 = mo_tab + pos_tab[2][None, :]
    cls0 = cls_token.reshape(1, H) + pos_tab[0][None, :]
    consts = (W1, W2, sptab0, motab0, posb, cls0, r(b1),
              r(ln1_g), r(ln1_b), r(ln2_g), r(ln2_b))
    g0 = sc(word_emb, idx4d[0]).reshape(BH, L, H)
    g1 = sc(word_emb, idx4d[1]).reshape(BH, L, H)

    out = _dense_call_h0(
        g0, values[:BH],
        species[:BH].reshape(STEPS_H, 1, G), modality[:BH].reshape(STEPS_H, 1, G),
        *consts)
    out = _dense_call_h1(
        g1, values[BH:],
        species[BH:].reshape(STEPS_H, 1, G), modality[BH:].reshape(STEPS_H, 1, G),
        *consts, out)
    return out


# speedup vs baseline: 1.7345x; 1.0200x over previous
import functools

import jax
import jax.numpy as jnp
from jax import lax
from jax.experimental import pallas as pl
from jax.experimental.pallas import tpu as pltpu
from jax.experimental.pallas import tpu_sc as plsc

B, L, H, V = 1024, 200, 128, 100000
P, T, S, M = 512, 2, 20, 4

NC, NS = 2, 16
NW = NC * NS
NHALF = 2
BH = B // NHALF
BLH = BH * L
ROWS_PER_W = BLH // NW
CPR = 128
NCHUNK = ROWS_PER_W // CPR


@functools.cache
def _sc_gather_call():
    mesh = plsc.VectorSubcoreMesh(core_axis_name="c", subcore_axis_name="s",
                                  num_cores=NC, num_subcores=NS)
    return pl.kernel(
        _sc_gather_body,
        out_type=jax.ShapeDtypeStruct((BLH, H), jnp.float32),
        mesh=mesh,
        scratch_types=[
            pltpu.VMEM((NCHUNK, CPR), jnp.int32),
            pltpu.VMEM((CPR, H), jnp.float32),
            pltpu.VMEM((CPR, H), jnp.float32),
            pltpu.SemaphoreType.DMA,
            pltpu.SemaphoreType.DMA,
        ],
    )


def _sc_gather_body(table_hbm, idx_hbm, out_hbm, idx_v, rows0, rows1, sem0,
                    sem1):
    wid = lax.axis_index("s") * NC + lax.axis_index("c")
    base = wid * ROWS_PER_W
    pltpu.sync_copy(idx_hbm.at[wid], idx_v)
    pltpu.async_copy(table_hbm.at[idx_v.at[0]], rows0, sem0)

    def body(t, _):
        a = 2 * t
        b = a + 1
        pltpu.async_copy(table_hbm.at[idx_v.at[b]], rows1, sem1)
        pltpu.make_async_copy(table_hbm.at[idx_v.at[a]], rows0, sem0).wait()
        pltpu.sync_copy(rows0, out_hbm.at[pl.ds(base + a * CPR, CPR)])

        @pl.when(b + 1 < NCHUNK)
        def _():
            pltpu.async_copy(table_hbm.at[idx_v.at[b + 1]], rows0, sem0)

        pltpu.make_async_copy(table_hbm.at[idx_v.at[b]], rows1, sem1).wait()
        pltpu.sync_copy(rows1, out_hbm.at[pl.ds(base + b * CPR, CPR)])
        return 0

    lax.fori_loop(0, NCHUNK // 2, body, 0)
    if NCHUNK % 2:
        j = NCHUNK - 1
        pltpu.make_async_copy(table_hbm.at[idx_v.at[j]], rows0, sem0).wait()
        pltpu.sync_copy(rows0, out_hbm.at[pl.ds(base + j * CPR, CPR)])


G = 64
STEPS_H = BH // G


def _ln_mxu(x2d, eps, g, b):
    jones = jnp.full((H, H), 1.0 / H, dtype=jnp.bfloat16)
    mu = jnp.dot(x2d.astype(jnp.bfloat16), jones,
                 preferred_element_type=jnp.float32)
    d = x2d - mu
    db = d.astype(jnp.bfloat16)
    var = jnp.dot(db * db, jones, preferred_element_type=jnp.float32)
    return d * lax.rsqrt(var + eps) * g + b


def _dense_body(g_ref, v_ref, sp_ref, mo_ref, w1_ref, w2_ref, sptab_ref,
                motab_ref, posb_ref, cls_ref, b1_ref,
                g1_ref, be1_ref, g2_ref, be2_ref, out_ref):
    e = g_ref[...].reshape(G * L, H).astype(jnp.bfloat16)
    h = jnp.dot(e, w1_ref[0:H, :].astype(jnp.bfloat16),
                preferred_element_type=jnp.float32).reshape(G, L, H)
    h = (h + v_ref[...][..., None] * w1_ref[H:H + 1, :][None]
         + b1_ref[...][None])
    h = _ln_mxu(h.reshape(G * L, H), 1e-5, g1_ref[...], be1_ref[...])
    h = h.astype(jnp.bfloat16)
    h = h * jax.nn.sigmoid(jnp.bfloat16(1.702) * h)
    h = jnp.dot(h, w2_ref[...].astype(jnp.bfloat16),
                preferred_element_type=jnp.float32).reshape(G, L, H)
    body3 = h + posb_ref[...][None]
    out_ref[:, 3:, :] = _ln_mxu(body3.reshape(G * L, H), 1e-12, g2_ref[...],
                                be2_ref[...]).reshape(G, L, H)

    spv = sp_ref[0, 0, :]
    mov = mo_ref[0, 0, :]
    sp_oh = (spv[:, None] == lax.broadcasted_iota(jnp.int32, (1, S), 1)
             ).astype(jnp.float32)
    mo_oh = (mov[:, None] == lax.broadcasted_iota(jnp.int32, (1, M), 1)
             ).astype(jnp.float32)
    sp_rows = jnp.dot(sp_oh, sptab_ref[...], preferred_element_type=jnp.float32)
    mo_rows = jnp.dot(mo_oh, motab_ref[...], preferred_element_type=jnp.float32)
    cls_rows = jnp.broadcast_to(cls_ref[...], (G, H))
    prefix = jnp.concatenate(
        [cls_rows[:, None, :], sp_rows[:, None, :], mo_rows[:, None, :]],
        axis=1)
    mu = jnp.mean(prefix, axis=-1, keepdims=True)
    var = jnp.mean((prefix - mu) ** 2, axis=-1, keepdims=True)
    out_ref[:, 0:3, :] = ((prefix - mu) * lax.rsqrt(var + 1e-12)
                          * g2_ref[...][None] + be2_ref[...][None])


def _whole(shape):
    n = len(shape)
    return pl.BlockSpec(shape, lambda i: (0,) * n)


def _dense_in_specs():
    return [
        pl.BlockSpec((G, L, H), lambda i: (i, 0, 0)),
        pl.BlockSpec((G, L), lambda i: (i, 0)),
        pl.BlockSpec((1, 1, G), lambda i: (i, 0, 0)),
        pl.BlockSpec((1, 1, G), lambda i: (i, 0, 0)),
        _whole((H + 1, H)),
        _whole((H, H)),
        _whole((S, H)),
        _whole((M, H)),
        _whole((L, H)),
        _whole((1, H)),
        _whole((1, H)),
        _whole((1, H)),
        _whole((1, H)),
        _whole((1, H)),
        _whole((1, H)),
    ]


_dense_call_h0 = pl.pallas_call(
    _dense_body,
    grid=(STEPS_H,),
    in_specs=_dense_in_specs(),
    out_specs=pl.BlockSpec((G, L + 3, H), lambda i: (i, 0, 0)),
    out_shape=jax.ShapeDtypeStruct((B, L + 3, H), jnp.float32),
)

def _dense_body_h1(*refs):
    _dense_body(*refs[:15], refs[16])


_dense_call_h1 = pl.pallas_call(
    _dense_body_h1,
    grid=(STEPS_H,),
    in_specs=_dense_in_specs() + [pl.BlockSpec(memory_space=pl.ANY)],
    out_specs=pl.BlockSpec((G, L + 3, H), lambda i: (i + STEPS_H, 0, 0)),
    out_shape=jax.ShapeDtypeStruct((B, L + 3, H), jnp.float32),
    input_output_aliases={15: 0},
)


def kernel(input_ids, values, species, modality, word_emb, cls_token,
           W1, b1, ln1_g, ln1_b, W2, b2, tte, pos_tab, sp_tab, mo_tab,
           ln2_g, ln2_b):
    sc = _sc_gather_call()
    idx4d = input_ids.reshape(NHALF, NW, NCHUNK, CPR)
    r = lambda a: a.reshape(1, H)
    posb = pos_tab[3:L + 3] + b2[None, :] + tte[0][None, :]
    sptab0 = sp_tab + pos_tab[1][None, :]
    motab0 = mo_tab + pos_tab[2][None, :]
    cls0 = cls_token.reshape(1, H) + pos_tab[0][None, :]
    consts = (W1, W2, sptab0, motab0, posb, cls0, r(b1),
              r(ln1_g), r(ln1_b), r(ln2_g), r(ln2_b))
    g0 = sc(word_emb, idx4d[0]).reshape(BH, L, H)
    g1 = sc(word_emb, idx4d[1]).reshape(BH, L, H)

    out = _dense_call_h0(
        g0, values[:BH],
        species[:BH].reshape(STEPS_H, 1, G), modality[:BH].reshape(STEPS_H, 1, G),
        *consts)
    out = _dense_call_h1(
        g1, values[BH:],
        species[BH:].reshape(STEPS_H, 1, G), modality[BH:].reshape(STEPS_H, 1, G),
        *consts, out)
    return out
